# Initial kernel scaffold; baseline (speedup 1.0000x reference)
import jax
import jax.numpy as jnp
from jax.experimental import pallas as pl


def kernel(inputs, Wq, keys_emb, values_emb, pk_Wq, pk_keys, Wout):
    raise NotImplementedError("write your pallas kernel here")


# speedup vs baseline: 3.6240x; 3.6240x over previous
import functools

import jax
import jax.numpy as jnp
from jax import lax
from jax.experimental import pallas as pl
from jax.experimental.pallas import tpu as pltpu

DIM = 768
HEADS = 12
NUM_KV = 784
NUM_KEYS = 28
TOPK = 8
DIM_KEY = 48
S = 2048
S_BLK_R = 512
S_BLK_G = 512
S_BLK_A = 256
PAD = 896

_NEG = -1e30


def _top8(c, width):
    j = lax.broadcasted_iota(jnp.int32, c.shape, 1)
    vals, idxs = [], []
    for _ in range(TOPK):
        m = jnp.max(c, axis=1, keepdims=True)
        sel = jnp.min(jnp.where(c == m, j, jnp.int32(1 << 30)), axis=1, keepdims=True)
        vals.append(m)
        idxs.append(sel)
        c = jnp.where(j == sel, _NEG, c)
    return jnp.concatenate(vals, axis=1), jnp.concatenate(idxs, axis=1)


def _router_body(x_ref, pkw0_ref, pkw1_ref, k0_ref, k1_ref, scores_ref, idx_ref):
    x = x_ref[...].astype(jnp.bfloat16)
    qpk0 = jnp.dot(x, pkw0_ref[0].astype(jnp.bfloat16), preferred_element_type=jnp.float32)
    qpk1 = jnp.dot(x, pkw1_ref[0].astype(jnp.bfloat16), preferred_element_type=jnp.float32)
    sim0 = jnp.dot(qpk0.astype(jnp.bfloat16), k0_ref[0].astype(jnp.bfloat16),
                   preferred_element_type=jnp.float32)
    sim1 = jnp.dot(qpk1.astype(jnp.bfloat16), k1_ref[0].astype(jnp.bfloat16),
                   preferred_element_type=jnp.float32)

    a_val, a_idx = _top8(sim0, NUM_KEYS)
    b_val, b_idx = _top8(sim1, NUM_KEYS)

    combos = jnp.concatenate(
        [a_val[:, ii:ii + 1] + b_val for ii in range(TOPK)], axis=1)
    scores, sel = _top8(combos, TOPK * TOPK)
    ii = sel // TOPK
    jj = sel % TOPK

    j8 = lax.broadcasted_iota(jnp.int32, (S_BLK_R, TOPK), 1)
    idx_cols = []
    for t in range(TOPK):
        ia = jnp.sum(jnp.where(j8 == ii[:, t:t + 1], a_idx, 0), axis=1, keepdims=True)
        ib = jnp.sum(jnp.where(j8 == jj[:, t:t + 1], b_idx, 0), axis=1, keepdims=True)
        idx_cols.append(ia + NUM_KEYS * ib)
    idx = jnp.concatenate(idx_cols, axis=1)

    m8 = jnp.max(scores, axis=1, keepdims=True)
    e = jnp.exp(scores - m8)
    p = e / jnp.sum(e, axis=1, keepdims=True)
    scores_ref[0] = p
    idx_ref[0] = idx


def _gather_body(idx_ref, scores_ref, kt_ref, vt_ref, k_ref, v_ref):
    kv_iota = lax.broadcasted_iota(jnp.int32, (S_BLK_G, NUM_KV), 1)
    a = jnp.zeros((S_BLK_G, NUM_KV), jnp.float32)
    idx = idx_ref[0]
    sc = scores_ref[0]
    for t in range(TOPK):
        a = a + jnp.where(kv_iota == idx[:, t:t + 1], sc[:, t:t + 1], 0.0)
    k_ref[0] = jnp.dot(a, kt_ref[0], preferred_element_type=jnp.float32)
    v_ref[0] = jnp.dot(a, vt_ref[0], preferred_element_type=jnp.float32)


def _attn_body(x_ref, wq_ref, k_ref, v_ref, wout_ref, out_ref, sim_ref):
    h = pl.program_id(0)
    qb = pl.program_id(1)
    nkb = qb + 1
    q = jnp.dot(x_ref[...], wq_ref[...], preferred_element_type=jnp.float32)
    q = q * (DIM ** -0.5)

    def qk_step(kb, _):
        kblk = k_ref[0, pl.ds(kb * S_BLK_A, S_BLK_A), :]
        sim_ref[:, pl.ds(kb * S_BLK_A, S_BLK_A)] = lax.dot_general(
            q, kblk, (((1,), (1,)), ((), ())), preferred_element_type=jnp.float32)
        return 0

    lax.fori_loop(0, nkb, qk_step, 0)

    col = lax.broadcasted_iota(jnp.int32, (S_BLK_A, S), 1)
    row = lax.broadcasted_iota(jnp.int32, (S_BLK_A, S), 0) + qb * S_BLK_A
    s = jnp.where(col > row, _NEG, sim_ref[...])
    m = jnp.max(s, axis=1, keepdims=True)
    p = jnp.exp(s - m)
    p = p / jnp.sum(p, axis=1, keepdims=True)
    sim_ref[...] = p

    def av_step(kb, o):
        pblk = sim_ref[:, pl.ds(kb * S_BLK_A, S_BLK_A)]
        vblk = v_ref[0, pl.ds(kb * S_BLK_A, S_BLK_A), :]
        return o + jnp.dot(pblk, vblk, preferred_element_type=jnp.float32)

    o = lax.fori_loop(0, nkb, av_step, jnp.zeros((S_BLK_A, DIM), jnp.float32))
    proj = jnp.dot(o, wout_ref[0], preferred_element_type=jnp.float32)

    @pl.when(h == 0)
    def _():
        out_ref[pl.ds(qb * S_BLK_A, S_BLK_A), :] = proj

    @pl.when(h > 0)
    def _():
        out_ref[pl.ds(qb * S_BLK_A, S_BLK_A), :] += proj


@jax.jit
def kernel(inputs, Wq, keys_emb, values_emb, pk_Wq, pk_keys, Wout):
    x = inputs[0]
    pk_keys_t = jnp.transpose(pk_keys, (0, 2, 3, 1))
    pkw = jnp.transpose(pk_Wq.reshape(DIM, 2, HEADS, DIM_KEY), (1, 2, 0, 3))

    scores, idx = pl.pallas_call(
        _router_body,
        grid=(HEADS, S // S_BLK_R),
        in_specs=[
            pl.BlockSpec((S_BLK_R, DIM), lambda h, sb: (sb, 0)),
            pl.BlockSpec((1, DIM, DIM_KEY), lambda h, sb: (h, 0, 0)),
            pl.BlockSpec((1, DIM, DIM_KEY), lambda h, sb: (h, 0, 0)),
            pl.BlockSpec((1, DIM_KEY, NUM_KEYS), lambda h, sb: (h, 0, 0)),
            pl.BlockSpec((1, DIM_KEY, NUM_KEYS), lambda h, sb: (h, 0, 0)),
        ],
        out_specs=[
            pl.BlockSpec((1, S_BLK_R, TOPK), lambda h, sb: (h, sb, 0)),
            pl.BlockSpec((1, S_BLK_R, TOPK), lambda h, sb: (h, sb, 0)),
        ],
        out_shape=[
            jax.ShapeDtypeStruct((HEADS, S, TOPK), jnp.float32),
            jax.ShapeDtypeStruct((HEADS, S, TOPK), jnp.int32),
        ],
    )(x, pkw[0], pkw[1], pk_keys_t[0], pk_keys_t[1])

    kt = keys_emb.reshape(HEADS, NUM_KV, DIM)
    vt = values_emb.reshape(HEADS, NUM_KV, DIM)
    k, v = pl.pallas_call(
        _gather_body,
        grid=(HEADS, S // S_BLK_G),
        in_specs=[
            pl.BlockSpec((1, S_BLK_G, TOPK), lambda h, sb: (h, sb, 0)),
            pl.BlockSpec((1, S_BLK_G, TOPK), lambda h, sb: (h, sb, 0)),
            pl.BlockSpec((1, NUM_KV, DIM), lambda h, sb: (h, 0, 0)),
            pl.BlockSpec((1, NUM_KV, DIM), lambda h, sb: (h, 0, 0)),
        ],
        out_specs=[
            pl.BlockSpec((1, S_BLK_G, DIM), lambda h, sb: (h, sb, 0)),
            pl.BlockSpec((1, S_BLK_G, DIM), lambda h, sb: (h, sb, 0)),
        ],
        out_shape=[
            jax.ShapeDtypeStruct((HEADS, S, DIM), jnp.float32),
            jax.ShapeDtypeStruct((HEADS, S, DIM), jnp.float32),
        ],
    )(idx, scores, kt, vt)

    wout3 = Wout.reshape(HEADS, DIM, DIM)
    out = pl.pallas_call(
        _attn_body,
        grid=(HEADS, S // S_BLK_A),
        in_specs=[
            pl.BlockSpec((S_BLK_A, DIM), lambda h, qb: (qb, 0)),
            pl.BlockSpec((DIM, DIM), lambda h, qb: (0, h)),
            pl.BlockSpec((1, S, DIM), lambda h, qb: (h, 0, 0)),
            pl.BlockSpec((1, S, DIM), lambda h, qb: (h, 0, 0)),
            pl.BlockSpec((1, DIM, DIM), lambda h, qb: (h, 0, 0)),
        ],
        out_specs=pl.BlockSpec((S, DIM), lambda h, qb: (0, 0)),
        out_shape=jax.ShapeDtypeStruct((S, DIM), jnp.float32),
        scratch_shapes=[pltpu.VMEM((S_BLK_A, S), jnp.float32)],
    )(x, Wq, k, v, wout3)

    return out[None]


# speedup vs baseline: 4.2856x; 1.1826x over previous
import jax
import jax.numpy as jnp
from jax import lax
from jax.experimental import pallas as pl
from jax.experimental.pallas import tpu as pltpu

DIM = 768
HEADS = 12
NUM_KV = 784
NUM_KEYS = 28
TOPK = 8
DIM_KEY = 48
S = 2048
S_BLK_R = 512
S_BLK_G = 512
S_BLK_A = 256
PAD = 896

_NEG = -1e30
_HIGHEST = lax.Precision.HIGHEST


def _top8(c):
    j = lax.broadcasted_iota(jnp.int32, c.shape, 1)
    vals, idxs = [], []
    for _ in range(TOPK):
        m = jnp.max(c, axis=1, keepdims=True)
        sel = jnp.min(jnp.where(c == m, j, jnp.int32(1 << 30)), axis=1, keepdims=True)
        vals.append(m)
        idxs.append(sel)
        c = jnp.where(j == sel, _NEG, c)
    return jnp.concatenate(vals, axis=1), jnp.concatenate(idxs, axis=1)


def _router_body(x_ref, pkw0_ref, pkw1_ref, k0_ref, k1_ref, scores_ref, idx_ref):
    x = x_ref[...].astype(jnp.bfloat16)
    qpk0 = jnp.dot(x, pkw0_ref[0], preferred_element_type=jnp.float32)
    qpk1 = jnp.dot(x, pkw1_ref[0], preferred_element_type=jnp.float32)
    sim0 = jnp.dot(qpk0.astype(jnp.bfloat16), k0_ref[0],
                   preferred_element_type=jnp.float32)
    sim1 = jnp.dot(qpk1.astype(jnp.bfloat16), k1_ref[0],
                   preferred_element_type=jnp.float32)

    r = lax.broadcasted_iota(jnp.int32, (NUM_KEYS, PAD), 0)
    j = lax.broadcasted_iota(jnp.int32, (NUM_KEYS, PAD), 1)
    valid = j < NUM_KV
    e0 = ((j % NUM_KEYS == r) & valid).astype(jnp.float32)
    e1 = ((j // NUM_KEYS == r) & valid).astype(jnp.float32)
    c = (jnp.dot(sim0, e0, precision=_HIGHEST, preferred_element_type=jnp.float32)
         + jnp.dot(sim1, e1, precision=_HIGHEST, preferred_element_type=jnp.float32))
    j2 = lax.broadcasted_iota(jnp.int32, (S_BLK_R, PAD), 1)
    c = jnp.where(j2 < NUM_KV, c, _NEG)

    scores, idx = _top8(c)
    m8 = jnp.max(scores, axis=1, keepdims=True)
    e = jnp.exp(scores - m8)
    p = e / jnp.sum(e, axis=1, keepdims=True)
    scores_ref[0] = p
    idx_ref[0] = idx


def _gather_body(idx_ref, scores_ref, kt_ref, vt_ref, k_ref, v_ref):
    kv_iota = lax.broadcasted_iota(jnp.int32, (S_BLK_G, NUM_KV), 1)
    a = jnp.zeros((S_BLK_G, NUM_KV), jnp.float32)
    idx = idx_ref[0]
    sc = scores_ref[0]
    for t in range(TOPK):
        a = a + jnp.where(kv_iota == idx[:, t:t + 1], sc[:, t:t + 1], 0.0)
    ab = a.astype(jnp.bfloat16)
    kt = kt_ref[...].astype(jnp.bfloat16)
    vt = vt_ref[...].astype(jnp.bfloat16)
    k_ref[0] = jnp.dot(ab, kt, preferred_element_type=jnp.float32).astype(jnp.bfloat16)
    v_ref[0] = jnp.dot(ab, vt, preferred_element_type=jnp.float32).astype(jnp.bfloat16)


def _attn_body(x_ref, wq_ref, k_ref, v_ref, wout_ref, out_ref, sim_ref):
    h = pl.program_id(0)
    qb = pl.program_id(1)
    nkb = qb + 1
    q = jnp.dot(x_ref[...].astype(jnp.bfloat16), wq_ref[...].astype(jnp.bfloat16),
                preferred_element_type=jnp.float32)
    q = (q * (DIM ** -0.5)).astype(jnp.bfloat16)

    def qk_step(kb, _):
        kblk = k_ref[0, pl.ds(kb * S_BLK_A, S_BLK_A), :]
        sim_ref[:, pl.ds(kb * S_BLK_A, S_BLK_A)] = lax.dot_general(
            q, kblk, (((1,), (1,)), ((), ())), preferred_element_type=jnp.float32)
        return 0

    lax.fori_loop(0, nkb, qk_step, 0)

    col = lax.broadcasted_iota(jnp.int32, (S_BLK_A, S), 1)
    row = lax.broadcasted_iota(jnp.int32, (S_BLK_A, S), 0) + qb * S_BLK_A
    s = jnp.where(col > row, _NEG, sim_ref[...])
    m = jnp.max(s, axis=1, keepdims=True)
    p = jnp.exp(s - m)
    p = p / jnp.sum(p, axis=1, keepdims=True)
    sim_ref[...] = p

    def av_step(kb, o):
        pblk = sim_ref[:, pl.ds(kb * S_BLK_A, S_BLK_A)].astype(jnp.bfloat16)
        vblk = v_ref[0, pl.ds(kb * S_BLK_A, S_BLK_A), :]
        return o + jnp.dot(pblk, vblk, preferred_element_type=jnp.float32)

    o = lax.fori_loop(0, nkb, av_step, jnp.zeros((S_BLK_A, DIM), jnp.float32))
    proj = jnp.dot(o.astype(jnp.bfloat16), wout_ref[...].astype(jnp.bfloat16),
                   preferred_element_type=jnp.float32)

    @pl.when(h == 0)
    def _():
        out_ref[pl.ds(qb * S_BLK_A, S_BLK_A), :] = proj

    @pl.when(h > 0)
    def _():
        out_ref[pl.ds(qb * S_BLK_A, S_BLK_A), :] += proj


@jax.jit
def kernel(inputs, Wq, keys_emb, values_emb, pk_Wq, pk_keys, Wout):
    x = inputs[0]
    pk_keys_t = jnp.transpose(pk_keys, (0, 2, 3, 1)).astype(jnp.bfloat16)
    pkw = jnp.transpose(pk_Wq.reshape(DIM, 2, HEADS, DIM_KEY),
                        (1, 2, 0, 3)).astype(jnp.bfloat16)

    scores, idx = pl.pallas_call(
        _router_body,
        grid=(HEADS, S // S_BLK_R),
        in_specs=[
            pl.BlockSpec((S_BLK_R, DIM), lambda h, sb: (sb, 0)),
            pl.BlockSpec((1, DIM, DIM_KEY), lambda h, sb: (h, 0, 0)),
            pl.BlockSpec((1, DIM, DIM_KEY), lambda h, sb: (h, 0, 0)),
            pl.BlockSpec((1, DIM_KEY, NUM_KEYS), lambda h, sb: (h, 0, 0)),
            pl.BlockSpec((1, DIM_KEY, NUM_KEYS), lambda h, sb: (h, 0, 0)),
        ],
        out_specs=[
            pl.BlockSpec((1, S_BLK_R, TOPK), lambda h, sb: (h, sb, 0)),
            pl.BlockSpec((1, S_BLK_R, TOPK), lambda h, sb: (h, sb, 0)),
        ],
        out_shape=[
            jax.ShapeDtypeStruct((HEADS, S, TOPK), jnp.float32),
            jax.ShapeDtypeStruct((HEADS, S, TOPK), jnp.int32),
        ],
    )(x, pkw[0], pkw[1], pk_keys_t[0], pk_keys_t[1])

    k, v = pl.pallas_call(
        _gather_body,
        grid=(HEADS, S // S_BLK_G),
        in_specs=[
            pl.BlockSpec((1, S_BLK_G, TOPK), lambda h, sb: (h, sb, 0)),
            pl.BlockSpec((1, S_BLK_G, TOPK), lambda h, sb: (h, sb, 0)),
            pl.BlockSpec((NUM_KV, DIM), lambda h, sb: (h, 0)),
            pl.BlockSpec((NUM_KV, DIM), lambda h, sb: (h, 0)),
        ],
        out_specs=[
            pl.BlockSpec((1, S_BLK_G, DIM), lambda h, sb: (h, sb, 0)),
            pl.BlockSpec((1, S_BLK_G, DIM), lambda h, sb: (h, sb, 0)),
        ],
        out_shape=[
            jax.ShapeDtypeStruct((HEADS, S, DIM), jnp.bfloat16),
            jax.ShapeDtypeStruct((HEADS, S, DIM), jnp.bfloat16),
        ],
    )(idx, scores, keys_emb, values_emb)

    out = pl.pallas_call(
        _attn_body,
        grid=(HEADS, S // S_BLK_A),
        in_specs=[
            pl.BlockSpec((S_BLK_A, DIM), lambda h, qb: (qb, 0)),
            pl.BlockSpec((DIM, DIM), lambda h, qb: (0, h)),
            pl.BlockSpec((1, S, DIM), lambda h, qb: (h, 0, 0)),
            pl.BlockSpec((1, S, DIM), lambda h, qb: (h, 0, 0)),
            pl.BlockSpec((DIM, DIM), lambda h, qb: (h, 0)),
        ],
        out_specs=pl.BlockSpec((S, DIM), lambda h, qb: (0, 0)),
        out_shape=jax.ShapeDtypeStruct((S, DIM), jnp.float32),
        scratch_shapes=[pltpu.VMEM((S_BLK_A, S), jnp.float32)],
    )(x, Wq, k, v, Wout)

    return out[None]
